# baseline (speedup 1.0000x reference)
import functools

import jax
import jax.numpy as jnp
from jax import lax
from jax.experimental import pallas as pl
from jax.experimental.pallas import tpu as pltpu

EPS = 1e-5
VMEM_LIMIT = 60000 * 1024
TILE = 2048


def _colsum(h):
    ones = jnp.ones((1, h.shape[0]), jnp.float32)
    return jnp.dot(ones, h, preferred_element_type=jnp.float32)


def _accum_stats(h, s_ref, ss_ref):
    @pl.when(pl.program_id(0) == 0)
    def _():
        s_ref[...] = jnp.zeros_like(s_ref)
        ss_ref[...] = jnp.zeros_like(ss_ref)

    s_ref[...] += _colsum(h)
    ss_ref[...] += _colsum(h * h)


def _fold(s, ss, gamma, beta, n):
    mean = s / n
    var = jnp.maximum(ss / n - mean * mean, 0.0)
    scale = gamma * lax.rsqrt(var + EPS)
    shift = beta - mean * scale
    return scale, shift


def _ab_kernel(x_ref, w1_ref, b1_ref, g1_ref, be1_ref,
               w2_ref, b2_ref, h2_ref, s_ref, ss_ref,
               g_ref, c_ref, sc_ref, sh_ref, *, n_rows):
    p = pl.program_id(0)
    j = pl.program_id(1)

    @pl.when(p == 0)
    def _phase0():
        xt = x_ref[...]

        @pl.when(j == 0)
        def _():
            g_ref[...] = jnp.zeros_like(g_ref)
            c_ref[...] = jnp.zeros_like(c_ref)

        g_ref[...] += lax.dot_general(xt, xt, (((0,), (0,)), ((), ())),
                                      preferred_element_type=jnp.float32)
        c_ref[...] += _colsum(xt)

    @pl.when(p == 1)
    def _phase1():
        @pl.when(j == 0)
        def _():
            w1 = w1_ref[...].astype(jnp.float32)
            b1 = b1_ref[...]
            u = jnp.dot(c_ref[...], w1, preferred_element_type=jnp.float32)
            s1 = u + n_rows * b1
            gw = jnp.dot(g_ref[...], w1, preferred_element_type=jnp.float32)
            ss1 = (jnp.sum(w1 * gw, axis=0, keepdims=True)
                   + 2.0 * b1 * u + n_rows * b1 * b1)
            sc, sh = _fold(s1, ss1, g1_ref[...], be1_ref[...], n_rows)
            sc_ref[...] = sc
            sh_ref[...] = sh
            s_ref[...] = jnp.zeros_like(s_ref)
            ss_ref[...] = jnp.zeros_like(ss_ref)

        h1 = jnp.dot(x_ref[...].astype(jnp.bfloat16), w1_ref[...],
                     preferred_element_type=jnp.float32) + b1_ref[...]
        a = jnp.maximum(h1 * sc_ref[...] + sh_ref[...], 0.0)
        h2 = jnp.dot(a.astype(jnp.bfloat16), w2_ref[...],
                     preferred_element_type=jnp.float32) + b2_ref[...]
        h2_ref[...] = h2.astype(jnp.bfloat16)
        s_ref[...] += _colsum(h2)
        ss_ref[...] += _colsum(h2 * h2)


def _cd_kernel(h2_ref, s2_ref, ss2_ref, g2_ref, be2_ref, w3_ref, b3_ref,
               g3_ref, be3_ref, x_ref, w4_ref, b4_ref, w5a_ref, w5b_ref,
               b5_ref, w6_ref, b6_ref, h3_ref, out_ref, s_ref, ss_ref,
               sc_ref, sh_ref, w5_ref, *, n_rows, n_feat):
    p = pl.program_id(0)
    j = pl.program_id(1)

    @pl.when(p == 0)
    def _phase0():
        @pl.when(j == 0)
        def _():
            sc, sh = _fold(s2_ref[...], ss2_ref[...], g2_ref[...],
                           be2_ref[...], n_rows)
            sc_ref[...] = sc
            sh_ref[...] = sh
            s_ref[...] = jnp.zeros_like(s_ref)
            ss_ref[...] = jnp.zeros_like(ss_ref)

        a = jnp.maximum(h2_ref[...].astype(jnp.float32) * sc_ref[...]
                        + sh_ref[...], 0.0)
        h3 = jnp.dot(a.astype(jnp.bfloat16), w3_ref[...],
                     preferred_element_type=jnp.float32) + b3_ref[...]
        h3_ref[...] = h3.astype(jnp.bfloat16)
        s_ref[...] += _colsum(h3)
        ss_ref[...] += _colsum(h3 * h3)

    @pl.when(p == 1)
    def _phase1():
        @pl.when(j == 0)
        def _():
            sc, sh = _fold(s_ref[...], ss_ref[...], g3_ref[...],
                           be3_ref[...], n_rows)
            sc_ref[...] = sc
            sh_ref[...] = sh
            w5_ref[:n_feat, :] = w5a_ref[...]
            w5_ref[n_feat:, :] = w5b_ref[...]

        a = jnp.maximum(h2_ref[...].astype(jnp.float32) * sc_ref[...]
                        + sh_ref[...], 0.0)
        emb = jnp.dot(a.astype(jnp.bfloat16), w4_ref[...],
                      preferred_element_type=jnp.float32) + b4_ref[...]
        xe = jnp.concatenate([x_ref[...].astype(jnp.bfloat16),
                              emb.astype(jnp.bfloat16)], axis=1)
        h5 = jnp.maximum(
            jnp.dot(xe, w5_ref[...], preferred_element_type=jnp.float32)
            + b5_ref[...], 0.0)
        out_ref[...] = (jnp.dot(h5.astype(jnp.bfloat16), w6_ref[...],
                                preferred_element_type=jnp.float32)
                        + b6_ref[...])


def _row_tiled(tile_n, cols):
    return pl.BlockSpec((tile_n, cols), lambda j: (j, 0))


def _resident(shape):
    return pl.BlockSpec(shape, lambda j: (0, 0))


def _resident2(shape):
    return pl.BlockSpec(shape, lambda p, j: (0, 0))


def _cparams():
    return pltpu.CompilerParams(
        dimension_semantics=("arbitrary",),
        vmem_limit_bytes=VMEM_LIMIT)


@jax.jit
def _forward(x, w1, b1, w2, b2, w3, b3, w4, b4, w5a, w5b, b5, w6, b6,
             g1, be1, g2, be2, g3, be3):
    n, f = x.shape
    h = w1.shape[1]
    e = w4.shape[1]
    o = w6.shape[1]
    nt = n // TILE
    nf = float(n)
    scsh = [pltpu.VMEM((1, h), jnp.float32), pltpu.VMEM((1, h), jnp.float32)]

    h2, s2, ss2 = pl.pallas_call(
        functools.partial(_ab_kernel, n_rows=nf),
        grid=(2, nt),
        in_specs=[pl.BlockSpec((TILE, f), lambda p, j: (j, 0)),
                  _resident2((f, h)), _resident2((1, h)),
                  _resident2((1, h)), _resident2((1, h)),
                  _resident2((h, h)), _resident2((1, h))],
        out_specs=(pl.BlockSpec((TILE, h),
                                lambda p, j: (jnp.where(p == 1, j, 0), 0)),
                   _resident2((1, h)), _resident2((1, h))),
        out_shape=(jax.ShapeDtypeStruct((n, h), jnp.bfloat16),
                   jax.ShapeDtypeStruct((1, h), jnp.float32),
                   jax.ShapeDtypeStruct((1, h), jnp.float32)),
        scratch_shapes=[pltpu.VMEM((f, f), jnp.float32),
                        pltpu.VMEM((1, f), jnp.float32)] + scsh,
        compiler_params=pltpu.CompilerParams(
            dimension_semantics=("arbitrary", "arbitrary"),
            vmem_limit_bytes=VMEM_LIMIT),
    )(x, w1.astype(jnp.bfloat16), b1, g1, be1,
      w2.astype(jnp.bfloat16), b2)

    _, out = pl.pallas_call(
        functools.partial(_cd_kernel, n_rows=nf, n_feat=f),
        grid=(2, nt),
        in_specs=[pl.BlockSpec((TILE, h), lambda p, j: (j, 0)),
                  _resident2((1, h)), _resident2((1, h)),
                  _resident2((1, h)), _resident2((1, h)),
                  _resident2((h, h)), _resident2((1, h)),
                  _resident2((1, h)), _resident2((1, h)),
                  pl.BlockSpec((TILE, f),
                               lambda p, j: (jnp.where(p == 1, j, 0), 0)),
                  _resident2((h, e)), _resident2((1, e)),
                  _resident2((f, h)), _resident2((e, h)), _resident2((1, h)),
                  _resident2((h, o)), _resident2((1, o))],
        out_specs=(pl.BlockSpec((TILE, h),
                                lambda p, j: (jnp.where(p == 0, j, 0), 0)),
                   pl.BlockSpec((TILE, o),
                                lambda p, j: (jnp.where(p == 1, j, 0), 0))),
        out_shape=(jax.ShapeDtypeStruct((n, h), jnp.bfloat16),
                   jax.ShapeDtypeStruct((n, o), jnp.float32)),
        scratch_shapes=scsh + scsh
        + [pltpu.VMEM((f + e, h), jnp.bfloat16)],
        input_output_aliases={0: 0},
        compiler_params=pltpu.CompilerParams(
            dimension_semantics=("arbitrary", "arbitrary"),
            vmem_limit_bytes=VMEM_LIMIT),
    )(h2, s2, ss2, g2, be2, w3.astype(jnp.bfloat16), b3, g3, be3, x,
      w4.astype(jnp.bfloat16), b4, w5a.astype(jnp.bfloat16),
      w5b.astype(jnp.bfloat16), b5, w6.astype(jnp.bfloat16), b6)
    return out


def kernel(x, w1, b1, w2, b2, w3, b3, w4, b4, w5a, w5b, b5, w6, b6,
           g1, be1, g2, be2, g3, be3):
    return _forward(x, w1, b1, w2, b2, w3, b3, w4, b4, w5a, w5b, b5,
                    w6, b6, g1, be1, g2, be2, g3, be3)


# speedup vs baseline: 1.1088x; 1.1088x over previous
import functools

import jax
import jax.numpy as jnp
from jax import lax
from jax.experimental import pallas as pl
from jax.experimental.pallas import tpu as pltpu

EPS = 1e-5
VMEM_LIMIT = 60000 * 1024
TILE = 2048
TILE_AB = 4096


def _colsum(h):
    ones = jnp.ones((1, h.shape[0]), jnp.float32)
    return jnp.dot(ones, h, preferred_element_type=jnp.float32)


def _accum_stats(h, s_ref, ss_ref):
    @pl.when(pl.program_id(0) == 0)
    def _():
        s_ref[...] = jnp.zeros_like(s_ref)
        ss_ref[...] = jnp.zeros_like(ss_ref)

    s_ref[...] += _colsum(h)
    ss_ref[...] += _colsum(h * h)


def _fold(s, ss, gamma, beta, n):
    mean = s / n
    var = jnp.maximum(ss / n - mean * mean, 0.0)
    scale = gamma * lax.rsqrt(var + EPS)
    shift = beta - mean * scale
    return scale, shift


def _ab_kernel(x_ref, w1_ref, b1_ref, g1_ref, be1_ref,
               w2_ref, b2_ref, h2_ref, s_ref, ss_ref,
               g_ref, c_ref, sc_ref, sh_ref, *, n_rows):
    p = pl.program_id(0)
    j = pl.program_id(1)

    @pl.when(p == 0)
    def _phase0():
        xt = x_ref[...]

        @pl.when(j == 0)
        def _():
            g_ref[...] = jnp.zeros_like(g_ref)
            c_ref[...] = jnp.zeros_like(c_ref)

        g_ref[...] += lax.dot_general(xt, xt, (((0,), (0,)), ((), ())),
                                      preferred_element_type=jnp.float32)
        c_ref[...] += _colsum(xt)

    @pl.when(p == 1)
    def _phase1():
        @pl.when(j == 0)
        def _():
            w1 = w1_ref[...]
            b1 = b1_ref[...]
            u = jnp.dot(c_ref[...], w1, preferred_element_type=jnp.float32)
            s1 = u + n_rows * b1
            gw = jnp.dot(g_ref[...], w1, preferred_element_type=jnp.float32)
            ss1 = (jnp.sum(w1 * gw, axis=0, keepdims=True)
                   + 2.0 * b1 * u + n_rows * b1 * b1)
            sc, sh = _fold(s1, ss1, g1_ref[...], be1_ref[...], n_rows)
            sc_ref[...] = sc
            sh_ref[...] = sh
            s_ref[...] = jnp.zeros_like(s_ref)
            ss_ref[...] = jnp.zeros_like(ss_ref)

        h1 = jnp.dot(x_ref[...], w1_ref[...],
                     preferred_element_type=jnp.float32) + b1_ref[...]
        a = jnp.maximum(h1 * sc_ref[...] + sh_ref[...], 0.0)
        h2 = jnp.dot(a, w2_ref[...],
                     preferred_element_type=jnp.float32) + b2_ref[...]
        h2_ref[...] = h2.astype(jnp.bfloat16)
        s_ref[...] += _colsum(h2)
        ss_ref[...] += _colsum(h2 * h2)


def _cd_kernel(h2_ref, s2_ref, ss2_ref, g2_ref, be2_ref, w3_ref, b3_ref,
               g3_ref, be3_ref, x_ref, w4_ref, b4_ref, w5a_ref, w5b_ref,
               b5_ref, w6_ref, b6_ref, h3_ref, out_ref, s_ref, ss_ref,
               sc_ref, sh_ref, w5_ref, *, n_rows, n_feat):
    p = pl.program_id(0)
    j = pl.program_id(1)

    @pl.when(p == 0)
    def _phase0():
        @pl.when(j == 0)
        def _():
            sc, sh = _fold(s2_ref[...], ss2_ref[...], g2_ref[...],
                           be2_ref[...], n_rows)
            sc_ref[...] = sc
            sh_ref[...] = sh
            s_ref[...] = jnp.zeros_like(s_ref)
            ss_ref[...] = jnp.zeros_like(ss_ref)

        a = jnp.maximum(h2_ref[...].astype(jnp.float32) * sc_ref[...]
                        + sh_ref[...], 0.0)
        h3 = jnp.dot(a, w3_ref[...],
                     preferred_element_type=jnp.float32) + b3_ref[...]
        h3_ref[...] = h3.astype(jnp.bfloat16)
        s_ref[...] += _colsum(h3)
        ss_ref[...] += _colsum(h3 * h3)

    @pl.when(p == 1)
    def _phase1():
        @pl.when(j == 0)
        def _():
            sc, sh = _fold(s_ref[...], ss_ref[...], g3_ref[...],
                           be3_ref[...], n_rows)
            sc_ref[...] = sc
            sh_ref[...] = sh
            w5_ref[:n_feat, :] = w5a_ref[...]
            w5_ref[n_feat:, :] = w5b_ref[...]

        a = jnp.maximum(h2_ref[...].astype(jnp.float32) * sc_ref[...]
                        + sh_ref[...], 0.0)
        emb = jnp.dot(a, w4_ref[...],
                      preferred_element_type=jnp.float32) + b4_ref[...]
        xe = jnp.concatenate([x_ref[...], emb], axis=1)
        h5 = jnp.maximum(
            jnp.dot(xe, w5_ref[...], preferred_element_type=jnp.float32)
            + b5_ref[...], 0.0)
        out_ref[...] = (jnp.dot(h5, w6_ref[...],
                                preferred_element_type=jnp.float32)
                        + b6_ref[...])


def _row_tiled(tile_n, cols):
    return pl.BlockSpec((tile_n, cols), lambda j: (j, 0))


def _resident(shape):
    return pl.BlockSpec(shape, lambda j: (0, 0))


def _resident2(shape):
    return pl.BlockSpec(shape, lambda p, j: (0, 0))


def _cparams():
    return pltpu.CompilerParams(
        dimension_semantics=("arbitrary",),
        vmem_limit_bytes=VMEM_LIMIT)


@jax.jit
def _forward(x, w1, b1, w2, b2, w3, b3, w4, b4, w5a, w5b, b5, w6, b6,
             g1, be1, g2, be2, g3, be3):
    n, f = x.shape
    h = w1.shape[1]
    e = w4.shape[1]
    o = w6.shape[1]
    nt = n // TILE
    nf = float(n)
    scsh = [pltpu.VMEM((1, h), jnp.float32), pltpu.VMEM((1, h), jnp.float32)]

    h2, s2, ss2 = pl.pallas_call(
        functools.partial(_ab_kernel, n_rows=nf),
        grid=(2, n // TILE_AB),
        in_specs=[pl.BlockSpec((TILE_AB, f), lambda p, j: (j, 0)),
                  _resident2((f, h)), _resident2((1, h)),
                  _resident2((1, h)), _resident2((1, h)),
                  _resident2((h, h)), _resident2((1, h))],
        out_specs=(pl.BlockSpec((TILE_AB, h),
                                lambda p, j: (jnp.where(p == 1, j, 0), 0)),
                   _resident2((1, h)), _resident2((1, h))),
        out_shape=(jax.ShapeDtypeStruct((n, h), jnp.bfloat16),
                   jax.ShapeDtypeStruct((1, h), jnp.float32),
                   jax.ShapeDtypeStruct((1, h), jnp.float32)),
        scratch_shapes=[pltpu.VMEM((f, f), jnp.float32),
                        pltpu.VMEM((1, f), jnp.float32)] + scsh,
        compiler_params=pltpu.CompilerParams(
            dimension_semantics=("arbitrary", "arbitrary"),
            vmem_limit_bytes=VMEM_LIMIT),
    )(x, w1, b1, g1, be1, w2, b2)

    _, out = pl.pallas_call(
        functools.partial(_cd_kernel, n_rows=nf, n_feat=f),
        grid=(2, nt),
        in_specs=[pl.BlockSpec((TILE, h), lambda p, j: (j, 0)),
                  _resident2((1, h)), _resident2((1, h)),
                  _resident2((1, h)), _resident2((1, h)),
                  _resident2((h, h)), _resident2((1, h)),
                  _resident2((1, h)), _resident2((1, h)),
                  pl.BlockSpec((TILE, f),
                               lambda p, j: (jnp.where(p == 1, j, 0), 0)),
                  _resident2((h, e)), _resident2((1, e)),
                  _resident2((f, h)), _resident2((e, h)), _resident2((1, h)),
                  _resident2((h, o)), _resident2((1, o))],
        out_specs=(pl.BlockSpec((TILE, h),
                                lambda p, j: (jnp.where(p == 0, j, 0), 0)),
                   pl.BlockSpec((TILE, o),
                                lambda p, j: (jnp.where(p == 1, j, 0), 0))),
        out_shape=(jax.ShapeDtypeStruct((n, h), jnp.bfloat16),
                   jax.ShapeDtypeStruct((n, o), jnp.float32)),
        scratch_shapes=scsh + scsh
        + [pltpu.VMEM((f + e, h), jnp.float32)],
        input_output_aliases={0: 0},
        compiler_params=pltpu.CompilerParams(
            dimension_semantics=("arbitrary", "arbitrary"),
            vmem_limit_bytes=VMEM_LIMIT),
    )(h2, s2, ss2, g2, be2, w3, b3, g3, be3, x, w4, b4, w5a, w5b, b5,
      w6, b6)
    return out


def kernel(x, w1, b1, w2, b2, w3, b3, w4, b4, w5a, w5b, b5, w6, b6,
           g1, be1, g2, be2, g3, be3):
    return _forward(x, w1, b1, w2, b2, w3, b3, w4, b4, w5a, w5b, b5,
                    w6, b6, g1, be1, g2, be2, g3, be3)


# speedup vs baseline: 1.1165x; 1.0069x over previous
import functools

import jax
import jax.numpy as jnp
from jax import lax
from jax.experimental import pallas as pl
from jax.experimental.pallas import tpu as pltpu

EPS = 1e-5
VMEM_LIMIT = 60000 * 1024
TILE = 2048
TILE_AB = 4096


def _colsum(h):
    ones = jnp.ones((1, h.shape[0]), jnp.float32)
    return jnp.dot(ones, h, preferred_element_type=jnp.float32)


def _fold(s, ss, gamma, beta, n):
    mean = s / n
    var = jnp.maximum(ss / n - mean * mean, 0.0)
    scale = gamma * lax.rsqrt(var + EPS)
    shift = beta - mean * scale
    return scale, shift


def _ab_kernel(x_ref, w1_ref, b1_ref, g1_ref, be1_ref,
               w2_ref, b2_ref, h2_ref, s_ref, ss_ref,
               g_ref, c_ref, sc_ref, sh_ref, *, n_rows):
    p = pl.program_id(0)
    j = pl.program_id(1)

    @pl.when(p == 0)
    def _phase0():
        xt = x_ref[...]

        @pl.when(j == 0)
        def _():
            g_ref[...] = jnp.zeros_like(g_ref)
            c_ref[...] = jnp.zeros_like(c_ref)

        g_ref[...] += lax.dot_general(xt, xt, (((0,), (0,)), ((), ())),
                                      preferred_element_type=jnp.float32)
        c_ref[...] += _colsum(xt)

    @pl.when(p == 1)
    def _phase1():
        @pl.when(j == 0)
        def _():
            w1 = w1_ref[...]
            b1 = b1_ref[...]
            u = jnp.dot(c_ref[...], w1, preferred_element_type=jnp.float32)
            s1 = u + n_rows * b1
            gw = jnp.dot(g_ref[...], w1, preferred_element_type=jnp.float32)
            ss1 = (jnp.sum(w1 * gw, axis=0, keepdims=True)
                   + 2.0 * b1 * u + n_rows * b1 * b1)
            sc, sh = _fold(s1, ss1, g1_ref[...], be1_ref[...], n_rows)
            sc_ref[...] = sc
            sh_ref[...] = sh
            s_ref[...] = jnp.zeros_like(s_ref)
            ss_ref[...] = jnp.zeros_like(ss_ref)

        h1 = jnp.dot(x_ref[...], w1_ref[...],
                     preferred_element_type=jnp.float32) + b1_ref[...]
        a = jnp.maximum(h1 * sc_ref[...] + sh_ref[...], 0.0)
        h2 = jnp.dot(a, w2_ref[...],
                     preferred_element_type=jnp.float32) + b2_ref[...]
        h2_ref[...] = h2.astype(jnp.bfloat16)
        s_ref[...] += _colsum(h2)
        ss_ref[...] += _colsum(h2 * h2)


def _cd_kernel(h2_ref, s2_ref, ss2_ref, g2_ref, be2_ref, w3_ref, b3_ref,
               g3_ref, be3_ref, x_ref, w4_ref, b4_ref, w5a_ref, w5b_ref,
               b5_ref, w6_ref, b6_ref, h3_ref, out_ref, s_ref, ss_ref,
               sc_ref, sh_ref, w5_ref, *, n_rows, n_feat):
    p = pl.program_id(0)
    j = pl.program_id(1)

    @pl.when(p == 0)
    def _phase0():
        @pl.when(j == 0)
        def _():
            sc, sh = _fold(s2_ref[...], ss2_ref[...], g2_ref[...],
                           be2_ref[...], n_rows)
            sc_ref[...] = sc
            sh_ref[...] = sh
            s_ref[...] = jnp.zeros_like(s_ref)
            ss_ref[...] = jnp.zeros_like(ss_ref)

        a = jnp.maximum(h2_ref[...].astype(jnp.float32) * sc_ref[...]
                        + sh_ref[...], 0.0)
        h3 = jnp.dot(a, w3_ref[...],
                     preferred_element_type=jnp.float32) + b3_ref[...]
        h3_ref[...] = h3.astype(jnp.bfloat16)
        s_ref[...] += _colsum(h3)
        ss_ref[...] += _colsum(h3 * h3)

    @pl.when(p == 1)
    def _phase1():
        @pl.when(j == 0)
        def _():
            sc, sh = _fold(s_ref[...], ss_ref[...], g3_ref[...],
                           be3_ref[...], n_rows)
            sc_ref[...] = sc
            sh_ref[...] = sh
            w5_ref[:n_feat, :] = w5a_ref[...]
            w5_ref[n_feat:, :] = w5b_ref[...]

        a = jnp.maximum(h2_ref[...].astype(jnp.float32) * sc_ref[...]
                        + sh_ref[...], 0.0)
        emb = jnp.dot(a, w4_ref[...],
                      preferred_element_type=jnp.float32) + b4_ref[...]
        xe = jnp.concatenate([x_ref[...], emb], axis=1)
        h5 = jnp.maximum(
            jnp.dot(xe, w5_ref[...], preferred_element_type=jnp.float32)
            + b5_ref[...], 0.0)
        out_ref[...] = (jnp.dot(h5, w6_ref[...],
                                preferred_element_type=jnp.float32)
                        + b6_ref[...])


def _resident2(shape):
    return pl.BlockSpec(shape, lambda p, j: (0, 0))


@jax.jit
def _forward(x, w1, b1, w2, b2, w3, b3, w4, b4, w5a, w5b, b5, w6, b6,
             g1, be1, g2, be2, g3, be3):
    n, f = x.shape
    h = w1.shape[1]
    e = w4.shape[1]
    o = w6.shape[1]
    nt = n // TILE
    nf = float(n)
    scsh = [pltpu.VMEM((1, h), jnp.float32), pltpu.VMEM((1, h), jnp.float32)]

    h2, s2, ss2 = pl.pallas_call(
        functools.partial(_ab_kernel, n_rows=nf),
        grid=(2, n // TILE_AB),
        in_specs=[pl.BlockSpec((TILE_AB, f), lambda p, j: (j, 0)),
                  _resident2((f, h)), _resident2((1, h)),
                  _resident2((1, h)), _resident2((1, h)),
                  _resident2((h, h)), _resident2((1, h))],
        out_specs=(pl.BlockSpec((TILE_AB, h),
                                lambda p, j: (jnp.where(p == 1, j, 0), 0)),
                   _resident2((1, h)), _resident2((1, h))),
        out_shape=(jax.ShapeDtypeStruct((n, h), jnp.bfloat16),
                   jax.ShapeDtypeStruct((1, h), jnp.float32),
                   jax.ShapeDtypeStruct((1, h), jnp.float32)),
        scratch_shapes=[pltpu.VMEM((f, f), jnp.float32),
                        pltpu.VMEM((1, f), jnp.float32)] + scsh,
        compiler_params=pltpu.CompilerParams(
            dimension_semantics=("arbitrary", "arbitrary"),
            vmem_limit_bytes=VMEM_LIMIT),
    )(x, w1, b1, g1, be1, w2, b2)

    _, out = pl.pallas_call(
        functools.partial(_cd_kernel, n_rows=nf, n_feat=f),
        grid=(2, nt),
        in_specs=[pl.BlockSpec((TILE, h), lambda p, j: (j, 0)),
                  _resident2((1, h)), _resident2((1, h)),
                  _resident2((1, h)), _resident2((1, h)),
                  _resident2((h, h)), _resident2((1, h)),
                  _resident2((1, h)), _resident2((1, h)),
                  pl.BlockSpec((TILE, f),
                               lambda p, j: (jnp.where(p == 1, j, 0), 0)),
                  _resident2((h, e)), _resident2((1, e)),
                  _resident2((f, h)), _resident2((e, h)), _resident2((1, h)),
                  _resident2((h, o)), _resident2((1, o))],
        out_specs=(pl.BlockSpec((TILE, h),
                                lambda p, j: (jnp.where(p == 0, j, 0), 0)),
                   pl.BlockSpec((TILE, o),
                                lambda p, j: (jnp.where(p == 1, j, 0), 0))),
        out_shape=(jax.ShapeDtypeStruct((n, h), jnp.bfloat16),
                   jax.ShapeDtypeStruct((n, o), jnp.float32)),
        scratch_shapes=scsh + scsh
        + [pltpu.VMEM((f + e, h), jnp.float32)],
        input_output_aliases={0: 0},
        compiler_params=pltpu.CompilerParams(
            dimension_semantics=("arbitrary", "arbitrary"),
            vmem_limit_bytes=VMEM_LIMIT),
    )(h2, s2, ss2, g2, be2, w3, b3, g3, be3, x, w4, b4, w5a, w5b, b5,
      w6, b6)
    return out


def kernel(x, w1, b1, w2, b2, w3, b3, w4, b4, w5a, w5b, b5, w6, b6,
           g1, be1, g2, be2, g3, be3):
    return _forward(x, w1, b1, w2, b2, w3, b3, w4, b4, w5a, w5b, b5,
                    w6, b6, g1, be1, g2, be2, g3, be3)
